# Initial kernel scaffold; baseline (speedup 1.0000x reference)
import jax
import jax.numpy as jnp
from jax import lax
from jax.experimental import pallas as pl
from jax.experimental.pallas import tpu as pltpu
from jax.experimental.pallas import tpu_sc as plsc

_NUM_FIELDS = 26
_VOCAB = 100000
_EMB = 16
_BATCH = 16384

_NC = 2
_NS = 16
_NW = _NC * _NS
_L = 16

_TOTAL = _BATCH * _NUM_FIELDS
_RPW = _TOTAL // _NW
_G = 128
_GPC = 13
_CHUNK = _G * _GPC
_NCHUNK = _RPW // _CHUNK


def _emb_kernel(tables_hbm, feats_hbm, out_hbm, feat_v, idx_v, rows_v, sems):
    wid = lax.axis_index("s") * _NC + lax.axis_index("c")
    base = wid * _RPW

    pltpu.sync_copy(feats_hbm.at[pl.ds(base, _RPW)], feat_v)

    lanes = lax.iota(jnp.int32, _L)

    def idx_body(i, _):
        off = i * _L
        g = base + off + lanes
        idx_v[pl.ds(off, _L)] = feat_v[pl.ds(off, _L)] + (g % _NUM_FIELDS) * _VOCAB
        return 0

    lax.fori_loop(0, _RPW // _L, idx_body, 0, unroll=8)

    def fire(chunk, buf):
        handles = []
        for g in range(_GPC):
            r0 = chunk * _CHUNK + g * _G
            handles.append(
                pltpu.async_copy(
                    tables_hbm.at[idx_v.at[pl.ds(r0, _G)]],
                    rows_v.at[buf, pl.ds(g * _G, _G)],
                    sems.at[buf],
                )
            )
        return handles

    inflight = fire(0, 0)
    for chunk in range(_NCHUNK):
        nxt = None
        if chunk + 1 < _NCHUNK:
            nxt = fire(chunk + 1, (chunk + 1) % 2)
        for h in inflight:
            h.wait()
        pltpu.sync_copy(
            rows_v.at[chunk % 2],
            out_hbm.at[pl.ds(base + chunk * _CHUNK, _CHUNK)],
        )
        inflight = nxt


@jax.jit
def _lookup(tables_flat, feats_flat):
    mesh = plsc.VectorSubcoreMesh(core_axis_name="c", subcore_axis_name="s")
    return pl.kernel(
        _emb_kernel,
        out_type=jax.ShapeDtypeStruct((_TOTAL, _EMB), jnp.float32),
        mesh=mesh,
        scratch_types=[
            pltpu.VMEM((_RPW,), jnp.int32),
            pltpu.VMEM((_RPW,), jnp.int32),
            pltpu.VMEM((2, _CHUNK, _EMB), jnp.float32),
            pltpu.SemaphoreType.DMA((2,)),
        ],
    )(tables_flat, feats_flat)


def kernel(features, tables):
    tables_flat = tables.reshape(_NUM_FIELDS * _VOCAB, _EMB)
    feats_flat = features.reshape(_TOTAL)
    out = _lookup(tables_flat, feats_flat)
    return out.reshape(_BATCH, _NUM_FIELDS * _EMB)


# speedup vs baseline: 1.1540x; 1.1540x over previous
import jax
import jax.numpy as jnp
from jax import lax
from jax.experimental import pallas as pl
from jax.experimental.pallas import tpu as pltpu
from jax.experimental.pallas import tpu_sc as plsc

_NUM_FIELDS = 26
_VOCAB = 100000
_EMB = 16
_BATCH = 16384

_NC = 2
_NS = 16
_NW = _NC * _NS
_L = 16

_TOTAL = _BATCH * _NUM_FIELDS
_RPW = _TOTAL // _NW
_G = 128
_GPC = 13
_CHUNK = _G * _GPC
_NCHUNK = _RPW // _CHUNK


def _emb_kernel(tables_hbm, feats_hbm, out_hbm, feat_v, idx_v, rows_v, sems):
    wid = lax.axis_index("s") * _NC + lax.axis_index("c")
    base = wid * _RPW

    pltpu.sync_copy(feats_hbm.at[pl.ds(base, _RPW)], feat_v)

    lanes = lax.iota(jnp.int32, _L)

    def idx_body(i, _):
        off = i * _L
        g = base + off + lanes
        idx_v[pl.ds(off, _L)] = feat_v[pl.ds(off, _L)] + (g % _NUM_FIELDS) * _VOCAB
        return 0

    lax.fori_loop(0, _RPW // _L, idx_body, 0, unroll=8)

    def fire(chunk, buf):
        handles = []
        for g in range(_GPC):
            r0 = chunk * _CHUNK + g * _G
            handles.append(
                pltpu.async_copy(
                    tables_hbm.at[idx_v.at[pl.ds(r0, _G)]],
                    rows_v.at[buf, pl.ds(g * _G, _G)],
                    sems.at[buf],
                )
            )
        return handles

    inflight = fire(0, 0)
    for chunk in range(_NCHUNK):
        nxt = None
        if chunk + 1 < _NCHUNK:
            nxt = fire(chunk + 1, (chunk + 1) % 2)
        for h in inflight:
            h.wait()
        pltpu.sync_copy(
            rows_v.at[chunk % 2],
            out_hbm.at[pl.ds(base + chunk * _CHUNK, _CHUNK)],
        )
        inflight = nxt


@jax.jit
def _lookup(tables_flat, feats_flat):
    mesh = plsc.VectorSubcoreMesh(core_axis_name="c", subcore_axis_name="s")
    return pl.kernel(
        _emb_kernel,
        out_type=jax.ShapeDtypeStruct((_TOTAL, _EMB), jnp.float32),
        mesh=mesh,
        scratch_types=[
            pltpu.VMEM((_RPW,), jnp.int32),
            pltpu.VMEM((_RPW,), jnp.int32),
            pltpu.VMEM((2, _CHUNK, _EMB), jnp.float32),
            pltpu.SemaphoreType.DMA((2,)),
        ],
        compiler_params=pltpu.CompilerParams(use_tc_tiling_on_sc=False),
    )(tables_flat, feats_flat)


def kernel(features, tables):
    tables_flat = tables.reshape(_NUM_FIELDS * _VOCAB, _EMB)
    feats_flat = features.reshape(_TOTAL)
    out = _lookup(tables_flat, feats_flat)
    return out.reshape(_BATCH, _NUM_FIELDS * _EMB)


# speedup vs baseline: 1.1570x; 1.0026x over previous
import jax
import jax.numpy as jnp
from jax import lax
from jax.experimental import pallas as pl
from jax.experimental.pallas import tpu as pltpu
from jax.experimental.pallas import tpu_sc as plsc

_NUM_FIELDS = 26
_VOCAB = 100000
_EMB = 16
_BATCH = 16384

_NC = 2
_NS = 16
_NW = _NC * _NS
_L = 16

_TOTAL = _BATCH * _NUM_FIELDS
_RPW = _TOTAL // _NW
_G = 128
_GPC = 13
_CHUNK = _G * _GPC
_NCHUNK = _RPW // _CHUNK
_NB = 4


def _emb_kernel(tables_hbm, feats_hbm, out_hbm, idx_v, rows_v, gsems, wsems):
    wid = lax.axis_index("s") * _NC + lax.axis_index("c")
    base = wid * _RPW

    pltpu.sync_copy(feats_hbm.at[pl.ds(base, _RPW)], idx_v)

    lanes = lax.iota(jnp.int32, _L)

    def idx_body(i, _):
        off = i * _L
        g = base + off + lanes
        idx_v[pl.ds(off, _L)] = idx_v[pl.ds(off, _L)] + (g % _NUM_FIELDS) * _VOCAB
        return 0

    lax.fori_loop(0, _RPW // _L, idx_body, 0, unroll=8)

    gather = {}
    wb = [None] * _NB

    def fire(chunk):
        buf = chunk % _NB
        if wb[buf] is not None:
            wb[buf].wait()
            wb[buf] = None
        handles = []
        for g in range(_GPC):
            r0 = chunk * _CHUNK + g * _G
            handles.append(
                pltpu.async_copy(
                    tables_hbm.at[idx_v.at[pl.ds(r0, _G)]],
                    rows_v.at[buf, pl.ds(g * _G, _G)],
                    gsems.at[buf],
                )
            )
        gather[chunk] = handles

    for chunk in range(min(_NB, _NCHUNK)):
        fire(chunk)
    for chunk in range(_NCHUNK):
        buf = chunk % _NB
        for h in gather.pop(chunk):
            h.wait()
        wb[buf] = pltpu.async_copy(
            rows_v.at[buf],
            out_hbm.at[pl.ds(base + chunk * _CHUNK, _CHUNK)],
            wsems.at[buf],
        )
        if chunk + _NB < _NCHUNK:
            fire(chunk + _NB)
    for w in wb:
        if w is not None:
            w.wait()


@jax.jit
def _lookup(tables_flat, feats_flat):
    mesh = plsc.VectorSubcoreMesh(core_axis_name="c", subcore_axis_name="s")
    return pl.kernel(
        _emb_kernel,
        out_type=jax.ShapeDtypeStruct((_TOTAL, _EMB), jnp.float32),
        mesh=mesh,
        scratch_types=[
            pltpu.VMEM((_RPW,), jnp.int32),
            pltpu.VMEM((_NB, _CHUNK, _EMB), jnp.float32),
            pltpu.SemaphoreType.DMA((_NB,)),
            pltpu.SemaphoreType.DMA((_NB,)),
        ],
        compiler_params=pltpu.CompilerParams(use_tc_tiling_on_sc=False),
    )(tables_flat, feats_flat)


def kernel(features, tables):
    tables_flat = tables.reshape(_NUM_FIELDS * _VOCAB, _EMB)
    feats_flat = features.reshape(_TOTAL)
    out = _lookup(tables_flat, feats_flat)
    return out.reshape(_BATCH, _NUM_FIELDS * _EMB)


# speedup vs baseline: 5.4512x; 4.7115x over previous
import jax
import jax.numpy as jnp
from jax import lax
from jax.experimental import pallas as pl
from jax.experimental.pallas import tpu as pltpu
from jax.experimental.pallas import tpu_sc as plsc

_NUM_FIELDS = 26
_VOCAB = 100000
_EMB = 16
_BATCH = 16384

_NC = 2
_NS = 16
_NW = _NC * _NS
_L = 16

_ROWS = _NUM_FIELDS * _EMB
_RPW = _ROWS // _NW
_BH = _BATCH // 2


def _emb_kernel(tt_hbm, ft_hbm, out_hbm, row_v, feat_v, out_v):
    wid = lax.axis_index("s") * _NC + lax.axis_index("c")

    for i in range(_RPW):
        r = i * _NW + wid
        f = r // _EMB
        pltpu.sync_copy(tt_hbm.at[r], row_v)
        for h in range(2):
            b0 = h * _BH
            pltpu.sync_copy(ft_hbm.at[f, pl.ds(b0, _BH)], feat_v)

            def gather_body(k, _):
                off = k * _L
                out_v[pl.ds(off, _L)] = plsc.load_gather(
                    row_v, [feat_v[pl.ds(off, _L)]]
                )
                return 0

            lax.fori_loop(0, _BH // _L, gather_body, 0, unroll=8)
            pltpu.sync_copy(out_v, out_hbm.at[r, pl.ds(b0, _BH)])


@jax.jit
def _lookup(tables_t, feats_t):
    mesh = plsc.VectorSubcoreMesh(core_axis_name="c", subcore_axis_name="s")
    return pl.kernel(
        _emb_kernel,
        out_type=jax.ShapeDtypeStruct((_ROWS, _BATCH), jnp.float32),
        mesh=mesh,
        scratch_types=[
            pltpu.VMEM((_VOCAB,), jnp.float32),
            pltpu.VMEM((_BH,), jnp.int32),
            pltpu.VMEM((_BH,), jnp.float32),
        ],
        compiler_params=pltpu.CompilerParams(
            use_tc_tiling_on_sc=True, needs_layout_passes=False
        ),
    )(tables_t, feats_t)


def kernel(features, tables):
    tables_t = tables.transpose(0, 2, 1).reshape(_ROWS, _VOCAB)
    feats_t = features.T
    out_t = _lookup(tables_t, feats_t)
    return out_t.T
